# baseline (speedup 1.0000x reference)
import functools

import jax
import jax.numpy as jnp
from jax import lax
from jax.experimental import pallas as pl
from jax.experimental.pallas import tpu as pltpu
from jax.experimental.pallas import tpu_sc as plsc

_VOCAB = 1000000
_EMB = 16
_CLS = 2

_ROWS_PER_BLK = 8192


def _project_body(e_ref, w_ref, b_ref, o0_ref, o1_ref):
    e = e_ref[...]
    o0_ref[...] = jnp.sum(e * w_ref[0:1, :], axis=1) + b_ref[0, 0]
    o1_ref[...] = jnp.sum(e * w_ref[1:2, :], axis=1) + b_ref[0, 1]


def _project_table(embed, W, b2d):
    grid = pl.cdiv(_VOCAB, _ROWS_PER_BLK)
    return pl.pallas_call(
        _project_body,
        grid=(grid,),
        in_specs=[
            pl.BlockSpec((_ROWS_PER_BLK, _EMB), lambda i: (i, 0)),
            pl.BlockSpec((_CLS, _EMB), lambda i: (0, 0)),
            pl.BlockSpec((1, _CLS), lambda i: (0, 0)),
        ],
        out_specs=[
            pl.BlockSpec((_ROWS_PER_BLK,), lambda i: (i,)),
            pl.BlockSpec((_ROWS_PER_BLK,), lambda i: (i,)),
        ],
        out_shape=[
            jax.ShapeDtypeStruct((_VOCAB,), jnp.float32),
            jax.ShapeDtypeStruct((_VOCAB,), jnp.float32),
        ],
    )(embed, W, b2d)


_CHUNK = 512


def _make_gather(n_total):
    info = plsc.get_sparse_core_info()
    nc, ns = info.num_cores, info.num_subcores
    nw = nc * ns
    per_w = n_total // nw
    steps = per_w // _CHUNK
    v_stage = _VOCAB // (ns // 2)
    mesh = plsc.VectorSubcoreMesh(core_axis_name="c", subcore_axis_name="s")

    @functools.partial(
        pl.kernel,
        out_type=jax.ShapeDtypeStruct((n_total * _CLS,), jnp.float32),
        mesh=mesh,
        scratch_types=[
            pltpu.VMEM_SHARED((_VOCAB,), jnp.float32),
            pltpu.VMEM_SHARED((_VOCAB,), jnp.float32),
            pltpu.VMEM((_CHUNK,), jnp.int32),
            pltpu.VMEM((_CHUNK,), jnp.float32),
            pltpu.VMEM((_CHUNK,), jnp.float32),
            pltpu.VMEM((_CHUNK * _CLS,), jnp.float32),
            pltpu.SemaphoreType.DMA,
        ],
        compiler_params=pltpu.CompilerParams(use_tc_tiling_on_sc=False,
                                             needs_layout_passes=False),
    )
    def gather_kernel(p0_hbm, p1_hbm, idx_hbm, out_hbm,
                      sh0, sh1, idx_v, v0, v1, ob, sem):
        cid = lax.axis_index("c")
        sid = lax.axis_index("s")
        wid = sid * nc + cid
        base = wid * per_w

        @pl.when(sid < ns // 2)
        def _stage0():
            pltpu.sync_copy(p0_hbm.at[pl.ds(sid * v_stage, v_stage)],
                            sh0.at[pl.ds(sid * v_stage, v_stage)])

        @pl.when(sid >= ns // 2)
        def _stage1():
            s2 = sid - ns // 2
            pltpu.sync_copy(p1_hbm.at[pl.ds(s2 * v_stage, v_stage)],
                            sh1.at[pl.ds(s2 * v_stage, v_stage)])

        plsc.subcore_barrier()

        def step(j, carry):
            off = base + j * _CHUNK
            pltpu.sync_copy(idx_hbm.at[pl.ds(off, _CHUNK)], idx_v)
            c0 = pltpu.async_copy(sh0.at[idx_v], v0, sem)
            c1 = pltpu.async_copy(sh1.at[idx_v], v1, sem)
            c0.wait()
            c1.wait()

            def inter(g, carry2):
                a = v0[pl.ds(g * 16, 16)]
                b = v1[pl.ds(g * 16, 16)]
                p_vec = lax.iota(jnp.int32, 16) * 2 + g * 32
                plsc.store_scatter(ob, [p_vec], a)
                plsc.store_scatter(ob, [p_vec + 1], b)
                return carry2

            lax.fori_loop(0, _CHUNK // 16, inter, 0)
            pltpu.sync_copy(ob, out_hbm.at[pl.ds(off * _CLS, _CHUNK * _CLS)])
            return carry

        lax.fori_loop(0, steps, step, 0)

    return gather_kernel


def kernel(x, embed, W, b):
    B, L = x.shape
    p0, p1 = _project_table(embed, W, b.reshape(1, _CLS))
    out = _make_gather(B * L)(p0, p1, x.reshape(-1))
    return out.reshape(B, L, _CLS)


# speedup vs baseline: 4.7919x; 4.7919x over previous
import functools

import jax
import jax.numpy as jnp
from jax import lax
from jax.experimental import pallas as pl
from jax.experimental.pallas import tpu as pltpu
from jax.experimental.pallas import tpu_sc as plsc

_VOCAB = 1000000
_EMB = 16
_CLS = 2

_ROWS_PER_BLK = 32768


def _project_body(e_ref, w_ref, b_ref, o0_ref, o1_ref):
    res = lax.dot_general(
        w_ref[...], e_ref[...],
        dimension_numbers=(((1,), (1,)), ((), ())),
        preferred_element_type=jnp.float32,
    ) + b_ref[...]
    o0_ref[...] = res[0, :]
    o1_ref[...] = res[1, :]


def _project_table(embed, W, b2d):
    grid = pl.cdiv(_VOCAB, _ROWS_PER_BLK)
    return pl.pallas_call(
        _project_body,
        grid=(grid,),
        in_specs=[
            pl.BlockSpec((_ROWS_PER_BLK, _EMB), lambda i: (i, 0)),
            pl.BlockSpec((_CLS, _EMB), lambda i: (0, 0)),
            pl.BlockSpec((_CLS, 1), lambda i: (0, 0)),
        ],
        out_specs=[
            pl.BlockSpec((_ROWS_PER_BLK,), lambda i: (i,)),
            pl.BlockSpec((_ROWS_PER_BLK,), lambda i: (i,)),
        ],
        out_shape=[
            jax.ShapeDtypeStruct((_VOCAB,), jnp.float32),
            jax.ShapeDtypeStruct((_VOCAB,), jnp.float32),
        ],
    )(embed, W, b2d)


_CHUNK = 640


def _make_gather(n_total):
    info = plsc.get_sparse_core_info()
    nc, ns = info.num_cores, info.num_subcores
    nw = nc * ns
    per_w = n_total // nw
    steps = per_w // _CHUNK
    v_stage = _VOCAB // (ns // 2)
    mesh = plsc.VectorSubcoreMesh(core_axis_name="c", subcore_axis_name="s")

    @functools.partial(
        pl.kernel,
        out_type=[
            jax.ShapeDtypeStruct((n_total,), jnp.float32),
            jax.ShapeDtypeStruct((n_total,), jnp.float32),
        ],
        mesh=mesh,
        scratch_types=[
            pltpu.VMEM_SHARED((_VOCAB,), jnp.float32),
            pltpu.VMEM_SHARED((_VOCAB,), jnp.float32),
            pltpu.VMEM((_CHUNK,), jnp.int32),
            pltpu.VMEM((_CHUNK,), jnp.float32),
            pltpu.VMEM((_CHUNK,), jnp.float32),
            pltpu.SemaphoreType.DMA,
        ],
        compiler_params=pltpu.CompilerParams(use_tc_tiling_on_sc=False,
                                             needs_layout_passes=False),
    )
    def gather_kernel(p0_hbm, p1_hbm, idx_hbm, o0_hbm, o1_hbm,
                      sh0, sh1, idx_v, v0, v1, sem):
        cid = lax.axis_index("c")
        sid = lax.axis_index("s")
        wid = sid * nc + cid
        base = wid * per_w

        @pl.when(sid < ns // 2)
        def _stage0():
            pltpu.sync_copy(p0_hbm.at[pl.ds(sid * v_stage, v_stage)],
                            sh0.at[pl.ds(sid * v_stage, v_stage)])

        @pl.when(sid >= ns // 2)
        def _stage1():
            s2 = sid - ns // 2
            pltpu.sync_copy(p1_hbm.at[pl.ds(s2 * v_stage, v_stage)],
                            sh1.at[pl.ds(s2 * v_stage, v_stage)])

        plsc.subcore_barrier()

        def step(j, carry):
            off = base + j * _CHUNK
            pltpu.sync_copy(idx_hbm.at[pl.ds(off, _CHUNK)], idx_v)
            c0 = pltpu.async_copy(sh0.at[idx_v], v0, sem)
            c1 = pltpu.async_copy(sh1.at[idx_v], v1, sem)
            c0.wait()
            c1.wait()
            pltpu.sync_copy(v0, o0_hbm.at[pl.ds(off, _CHUNK)])
            pltpu.sync_copy(v1, o1_hbm.at[pl.ds(off, _CHUNK)])
            return carry

        lax.fori_loop(0, steps, step, 0)

    return gather_kernel


def kernel(x, embed, W, b):
    B, L = x.shape
    p0, p1 = _project_table(embed, W, b.reshape(_CLS, 1))
    o0, o1 = _make_gather(B * L)(p0, p1, x.reshape(-1))
    return jnp.stack([o0, o1], axis=-1).reshape(B, L, _CLS)
